# baseline (speedup 1.0000x reference)
import functools

import jax
import jax.numpy as jnp
from jax import lax
from jax.experimental import pallas as pl
from jax.experimental.pallas import tpu as pltpu
from jax.experimental.pallas import tpu_sc as plsc

EMB = 64
HID = 128
NC, NS = 2, 16
NW = NC * NS
CHUNK = 100


def _sc_gather(table, idx3):
    K = idx3.shape[1]
    mesh = plsc.VectorSubcoreMesh(core_axis_name="c", subcore_axis_name="s")

    @functools.partial(
        pl.kernel,
        out_type=jax.ShapeDtypeStruct((NW, K, CHUNK, EMB), jnp.bfloat16),
        mesh=mesh,
        scratch_types=[
            pltpu.VMEM((K, CHUNK), jnp.int32),
            pltpu.VMEM((K, CHUNK, EMB), jnp.bfloat16),
            pltpu.SemaphoreType.DMA,
        ],
        compiler_params=pltpu.CompilerParams(use_tc_tiling_on_sc=False),
    )
    def gather_kernel(table_hbm, idx_hbm, out_hbm, idx_v, rows_v, sem):
        wid = lax.axis_index("s") * NC + lax.axis_index("c")
        pltpu.sync_copy(idx_hbm.at[wid], idx_v)
        copies = [
            pltpu.async_copy(table_hbm.at[idx_v.at[j]], rows_v.at[j], sem)
            for j in range(K)
        ]
        for c in copies:
            c.wait()
        pltpu.sync_copy(rows_v, out_hbm.at[wid])

    return gather_kernel(table, idx3)


def _gru_body(xs_ref, wih_ref, whh_ref, bih_ref, bhh_ref, out_ref, h_ref):
    t = pl.program_id(0)

    @pl.when(t == 0)
    def _():
        h_ref[...] = jnp.zeros_like(h_ref)

    x = xs_ref[0]
    h = h_ref[...]
    gi = jnp.dot(x, wih_ref[...], preferred_element_type=jnp.float32) + bih_ref[...]
    gh = jnp.dot(h.astype(jnp.bfloat16), whh_ref[...],
                 preferred_element_type=jnp.float32) + bhh_ref[...]
    r = jax.nn.sigmoid(gi[:, :HID] + gh[:, :HID])
    z = jax.nn.sigmoid(gi[:, HID:2 * HID] + gh[:, HID:2 * HID])
    n = jnp.tanh(gi[:, 2 * HID:] + r * gh[:, 2 * HID:])
    h_new = n + z * (h - n)
    h_ref[...] = h_new

    @pl.when(t == pl.num_programs(0) - 1)
    def _():
        out_ref[0] = h_new


def _gru(xs, wih_t, whh_t, bih, bhh, interpret=False):
    T, B, _ = xs.shape
    return pl.pallas_call(
        _gru_body,
        grid=(T,),
        in_specs=[
            pl.BlockSpec((1, B, EMB), lambda t: (t, 0, 0)),
            pl.BlockSpec((EMB, 3 * HID), lambda t: (0, 0)),
            pl.BlockSpec((HID, 3 * HID), lambda t: (0, 0)),
            pl.BlockSpec((1, 3 * HID), lambda t: (0, 0)),
            pl.BlockSpec((1, 3 * HID), lambda t: (0, 0)),
        ],
        out_specs=pl.BlockSpec((1, B, HID), lambda t: (0, 0, 0)),
        out_shape=jax.ShapeDtypeStruct((1, B, HID), jnp.float32),
        scratch_shapes=[pltpu.VMEM((B, HID), jnp.float32)],
        interpret=interpret,
    )(xs, wih_t, whh_t, bih, bhh)


def kernel(src, emb_table, W_ih, W_hh, b_ih, b_hh):
    B, T = src.shape
    n = B * T
    per_w = n // NW
    k = per_w // CHUNK
    idx3 = src.astype(jnp.int32).T.reshape(NW, k, CHUNK)
    tab16 = emb_table.astype(jnp.bfloat16)
    rows = _sc_gather(tab16, idx3)
    xs = rows.reshape(T, B, EMB)
    h = _gru(xs, W_ih.T.astype(jnp.bfloat16), W_hh.T.astype(jnp.bfloat16),
             b_ih.reshape(1, -1), b_hh.reshape(1, -1))
    return h


# speedup vs baseline: 1.2347x; 1.2347x over previous
import functools

import jax
import jax.numpy as jnp
from jax import lax
from jax.experimental import pallas as pl
from jax.experimental.pallas import tpu as pltpu
from jax.experimental.pallas import tpu_sc as plsc

EMB = 64
HID = 128
NC, NS = 2, 16
NW = NC * NS
CHUNK = 100


def _sc_gather(table, idx3):
    K = idx3.shape[1]
    mesh = plsc.VectorSubcoreMesh(core_axis_name="c", subcore_axis_name="s")

    @functools.partial(
        pl.kernel,
        out_type=jax.ShapeDtypeStruct((NW, K, CHUNK, EMB), jnp.float32),
        mesh=mesh,
        scratch_types=[
            pltpu.VMEM((K, CHUNK), jnp.int32),
            pltpu.VMEM((K, CHUNK, EMB), jnp.float32),
            pltpu.SemaphoreType.DMA,
        ],
        compiler_params=pltpu.CompilerParams(use_tc_tiling_on_sc=False),
    )
    def gather_kernel(table_hbm, idx_hbm, out_hbm, idx_v, rows_v, sem):
        wid = lax.axis_index("s") * NC + lax.axis_index("c")
        pltpu.sync_copy(idx_hbm.at[wid], idx_v)
        copies = [
            pltpu.async_copy(table_hbm.at[idx_v.at[j]], rows_v.at[j], sem)
            for j in range(K)
        ]
        for c in copies:
            c.wait()
        pltpu.sync_copy(rows_v, out_hbm.at[wid])

    return gather_kernel(table, idx3)


def _transpose_body(eye_ref, in_ref, out_ref):
    out_ref[...] = lax.dot_general(
        in_ref[...], eye_ref[...], (((0,), (0,)), ((), ())),
        preferred_element_type=jnp.float32)


def _transpose_table(tabT, vb=8192):
    V = tabT.shape[1]
    eye = jnp.eye(EMB, dtype=jnp.float32)
    return pl.pallas_call(
        _transpose_body,
        grid=(V // vb,),
        in_specs=[
            pl.BlockSpec((EMB, EMB), lambda i: (0, 0)),
            pl.BlockSpec((EMB, vb), lambda i: (0, i)),
        ],
        out_specs=pl.BlockSpec((vb, EMB), lambda i: (i, 0)),
        out_shape=jax.ShapeDtypeStruct((V, EMB), jnp.float32),
    )(eye, tabT)


def _gru_body(xs_ref, wih_ref, whh_ref, bih_ref, bhh_ref, out_ref, h_ref):
    t = pl.program_id(0)

    @pl.when(t == 0)
    def _():
        h_ref[...] = jnp.zeros_like(h_ref)

    x = xs_ref[0]
    h = h_ref[...]
    gi = jnp.dot(x, wih_ref[...], preferred_element_type=jnp.float32) + bih_ref[...]
    gh = jnp.dot(h, whh_ref[...], preferred_element_type=jnp.float32) + bhh_ref[...]
    r = jax.nn.sigmoid(gi[:, :HID] + gh[:, :HID])
    z = jax.nn.sigmoid(gi[:, HID:2 * HID] + gh[:, HID:2 * HID])
    n = jnp.tanh(gi[:, 2 * HID:] + r * gh[:, 2 * HID:])
    h_new = n + z * (h - n)
    h_ref[...] = h_new

    @pl.when(t == pl.num_programs(0) - 1)
    def _():
        out_ref[0] = h_new


def _gru(xs, wih_t, whh_t, bih, bhh, interpret=False):
    T, B, _ = xs.shape
    return pl.pallas_call(
        _gru_body,
        grid=(T,),
        in_specs=[
            pl.BlockSpec((1, B, EMB), lambda t: (t, 0, 0)),
            pl.BlockSpec((EMB, 3 * HID), lambda t: (0, 0)),
            pl.BlockSpec((HID, 3 * HID), lambda t: (0, 0)),
            pl.BlockSpec((1, 3 * HID), lambda t: (0, 0)),
            pl.BlockSpec((1, 3 * HID), lambda t: (0, 0)),
        ],
        out_specs=pl.BlockSpec((1, B, HID), lambda t: (0, 0, 0)),
        out_shape=jax.ShapeDtypeStruct((1, B, HID), jnp.float32),
        scratch_shapes=[pltpu.VMEM((B, HID), jnp.float32)],
        interpret=interpret,
    )(xs, wih_t, whh_t, bih, bhh)


def kernel(src, emb_table, W_ih, W_hh, b_ih, b_hh):
    B, T = src.shape
    n = B * T
    per_w = n // NW
    k = per_w // CHUNK
    idx3 = src.astype(jnp.int32).T.reshape(NW, k, CHUNK)
    tab16 = _transpose_table(emb_table.T)
    rows = _sc_gather(tab16, idx3)
    xs = rows.reshape(T, B, EMB)
    h = _gru(xs, W_ih.T, W_hh.T, b_ih.reshape(1, -1), b_hh.reshape(1, -1))
    return h


# speedup vs baseline: 1.2459x; 1.0091x over previous
import functools

import jax
import jax.numpy as jnp
from jax import lax
from jax.experimental import pallas as pl
from jax.experimental.pallas import tpu as pltpu
from jax.experimental.pallas import tpu_sc as plsc

EMB = 64
HID = 128
NC, NS = 2, 16
NW = NC * NS
CHUNK = 100


def _sc_gather(table, idx3):
    K = idx3.shape[1]
    mesh = plsc.VectorSubcoreMesh(core_axis_name="c", subcore_axis_name="s")

    @functools.partial(
        pl.kernel,
        out_type=jax.ShapeDtypeStruct((NW, K, CHUNK, EMB), jnp.float32),
        mesh=mesh,
        scratch_types=[
            pltpu.VMEM((K, CHUNK), jnp.int32),
            pltpu.VMEM((K, CHUNK, EMB), jnp.float32),
            pltpu.SemaphoreType.DMA,
        ],
        compiler_params=pltpu.CompilerParams(use_tc_tiling_on_sc=False),
    )
    def gather_kernel(table_hbm, idx_hbm, out_hbm, idx_v, rows_v, sem):
        wid = lax.axis_index("s") * NC + lax.axis_index("c")
        pltpu.sync_copy(idx_hbm.at[wid], idx_v)
        copies = [
            pltpu.async_copy(table_hbm.at[idx_v.at[j]], rows_v.at[j], sem)
            for j in range(K)
        ]
        for c in copies:
            c.wait()
        pltpu.sync_copy(rows_v, out_hbm.at[wid])

    return gather_kernel(table, idx3)


def _transpose_body(in_ref, out_ref):
    out_ref[...] = in_ref[...].T


def _transpose_table(tabT, vb=8192):
    V = tabT.shape[1]
    return pl.pallas_call(
        _transpose_body,
        grid=(V // vb,),
        in_specs=[pl.BlockSpec((EMB, vb), lambda i: (0, i))],
        out_specs=pl.BlockSpec((vb, EMB), lambda i: (i, 0)),
        out_shape=jax.ShapeDtypeStruct((V, EMB), jnp.float32),
    )(tabT)


def _gru_body(xs_ref, wih_ref, whh_ref, bih_ref, bhh_ref, out_ref, h_ref):
    t = pl.program_id(0)

    @pl.when(t == 0)
    def _():
        h_ref[...] = jnp.zeros_like(h_ref)

    x = xs_ref[0]
    h = h_ref[...]
    gi = jnp.dot(x, wih_ref[...], preferred_element_type=jnp.float32) + bih_ref[...]
    gh = jnp.dot(h, whh_ref[...], preferred_element_type=jnp.float32) + bhh_ref[...]
    r = jax.nn.sigmoid(gi[:, :HID] + gh[:, :HID])
    z = jax.nn.sigmoid(gi[:, HID:2 * HID] + gh[:, HID:2 * HID])
    n = jnp.tanh(gi[:, 2 * HID:] + r * gh[:, 2 * HID:])
    h_new = n + z * (h - n)
    h_ref[...] = h_new

    @pl.when(t == pl.num_programs(0) - 1)
    def _():
        out_ref[0] = h_new


def _gru(xs, wih_t, whh_t, bih, bhh, interpret=False):
    T, B, _ = xs.shape
    return pl.pallas_call(
        _gru_body,
        grid=(T,),
        in_specs=[
            pl.BlockSpec((1, B, EMB), lambda t: (t, 0, 0)),
            pl.BlockSpec((EMB, 3 * HID), lambda t: (0, 0)),
            pl.BlockSpec((HID, 3 * HID), lambda t: (0, 0)),
            pl.BlockSpec((1, 3 * HID), lambda t: (0, 0)),
            pl.BlockSpec((1, 3 * HID), lambda t: (0, 0)),
        ],
        out_specs=pl.BlockSpec((1, B, HID), lambda t: (0, 0, 0)),
        out_shape=jax.ShapeDtypeStruct((1, B, HID), jnp.float32),
        scratch_shapes=[pltpu.VMEM((B, HID), jnp.float32)],
        interpret=interpret,
    )(xs, wih_t, whh_t, bih, bhh)


def kernel(src, emb_table, W_ih, W_hh, b_ih, b_hh):
    B, T = src.shape
    n = B * T
    per_w = n // NW
    k = per_w // CHUNK
    idx3 = src.astype(jnp.int32).T.reshape(NW, k, CHUNK)
    tab16 = _transpose_table(emb_table.T)
    rows = _sc_gather(tab16, idx3)
    xs = rows.reshape(T, B, EMB)
    h = _gru(xs, W_ih.T, W_hh.T, b_ih.reshape(1, -1), b_hh.reshape(1, -1))
    return h


# speedup vs baseline: 1.2499x; 1.0032x over previous
import functools

import jax
import jax.numpy as jnp
from jax import lax
from jax.experimental import pallas as pl
from jax.experimental.pallas import tpu as pltpu
from jax.experimental.pallas import tpu_sc as plsc

EMB = 64
HID = 128
NC, NS = 2, 16
NW = NC * NS
CHUNK = 100


def _sc_gather(table, idx3):
    K = idx3.shape[1]
    mesh = plsc.VectorSubcoreMesh(core_axis_name="c", subcore_axis_name="s")

    @functools.partial(
        pl.kernel,
        out_type=jax.ShapeDtypeStruct((NW, K, CHUNK, EMB), jnp.float32),
        mesh=mesh,
        scratch_types=[
            pltpu.VMEM((K, CHUNK), jnp.int32),
            pltpu.VMEM((K, CHUNK, EMB), jnp.float32),
            pltpu.SemaphoreType.DMA,
        ],
        compiler_params=pltpu.CompilerParams(use_tc_tiling_on_sc=False),
    )
    def gather_kernel(table_hbm, idx_hbm, out_hbm, idx_v, rows_v, sem):
        wid = lax.axis_index("s") * NC + lax.axis_index("c")
        pltpu.sync_copy(idx_hbm.at[wid], idx_v)
        copies = [
            pltpu.async_copy(table_hbm.at[idx_v.at[j]], rows_v.at[j], sem)
            for j in range(K)
        ]
        for c in copies:
            c.wait()
        pltpu.sync_copy(rows_v, out_hbm.at[wid])

    return gather_kernel(table, idx3)


def _transpose_body(in_ref, out_ref):
    out_ref[...] = in_ref[...].T


def _transpose_table(tabT, vb=8192):
    V = tabT.shape[1]
    return pl.pallas_call(
        _transpose_body,
        grid=(pl.cdiv(V, vb),),
        in_specs=[pl.BlockSpec((EMB, vb), lambda i: (0, i))],
        out_specs=pl.BlockSpec((vb, EMB), lambda i: (i, 0)),
        out_shape=jax.ShapeDtypeStruct((V, EMB), jnp.float32),
    )(tabT)


def _gru_body(xs_ref, wih_ref, whh_ref, bih_ref, bhh_ref, out_ref, h_ref):
    t = pl.program_id(0)

    @pl.when(t == 0)
    def _():
        h_ref[...] = jnp.zeros_like(h_ref)

    x = xs_ref[0]
    h = h_ref[...]
    gi = jnp.dot(x, wih_ref[...], preferred_element_type=jnp.float32) + bih_ref[...]
    gh = jnp.dot(h, whh_ref[...], preferred_element_type=jnp.float32) + bhh_ref[...]
    r = jax.nn.sigmoid(gi[:, :HID] + gh[:, :HID])
    z = jax.nn.sigmoid(gi[:, HID:2 * HID] + gh[:, HID:2 * HID])
    n = jnp.tanh(gi[:, 2 * HID:] + r * gh[:, 2 * HID:])
    h_new = n + z * (h - n)
    h_ref[...] = h_new

    @pl.when(t == pl.num_programs(0) - 1)
    def _():
        out_ref[0] = h_new


def _gru(xs, wih_t, whh_t, bih, bhh, interpret=False):
    T, B, _ = xs.shape
    return pl.pallas_call(
        _gru_body,
        grid=(T,),
        in_specs=[
            pl.BlockSpec((1, B, EMB), lambda t: (t, 0, 0)),
            pl.BlockSpec((EMB, 3 * HID), lambda t: (0, 0)),
            pl.BlockSpec((HID, 3 * HID), lambda t: (0, 0)),
            pl.BlockSpec((1, 3 * HID), lambda t: (0, 0)),
            pl.BlockSpec((1, 3 * HID), lambda t: (0, 0)),
        ],
        out_specs=pl.BlockSpec((1, B, HID), lambda t: (0, 0, 0)),
        out_shape=jax.ShapeDtypeStruct((1, B, HID), jnp.float32),
        scratch_shapes=[pltpu.VMEM((B, HID), jnp.float32)],
        interpret=interpret,
    )(xs, wih_t, whh_t, bih, bhh)


def kernel(src, emb_table, W_ih, W_hh, b_ih, b_hh):
    B, T = src.shape
    n = B * T
    per_w = n // NW
    k = per_w // CHUNK
    idx3 = src.astype(jnp.int32).T.reshape(NW, k, CHUNK)
    tab16 = _transpose_table(emb_table.T)
    rows = _sc_gather(tab16, idx3)
    xs = rows.reshape(T, B, EMB)
    h = _gru(xs, W_ih.T, W_hh.T, b_ih.reshape(1, -1), b_hh.reshape(1, -1))
    return h


# speedup vs baseline: 2.6326x; 2.1063x over previous
import functools

import jax
import jax.numpy as jnp
from jax import lax
from jax.experimental import pallas as pl
from jax.experimental.pallas import tpu as pltpu
from jax.experimental.pallas import tpu_sc as plsc

EMB = 64
HID = 128
NC, NS = 2, 16
NW = NC * NS
CHUNK = 100


def _sc_gather(table, idx3):
    K = idx3.shape[1]
    mesh = plsc.VectorSubcoreMesh(core_axis_name="c", subcore_axis_name="s")

    @functools.partial(
        pl.kernel,
        out_type=jax.ShapeDtypeStruct((NW, K, CHUNK, EMB), jnp.float32),
        mesh=mesh,
        scratch_types=[
            pltpu.VMEM((K, CHUNK), jnp.int32),
            pltpu.VMEM((K, CHUNK, EMB), jnp.float32),
            pltpu.SemaphoreType.DMA,
        ],
        compiler_params=pltpu.CompilerParams(use_tc_tiling_on_sc=False),
    )
    def gather_kernel(table_hbm, idx_hbm, out_hbm, idx_v, rows_v, sem):
        wid = lax.axis_index("s") * NC + lax.axis_index("c")
        pltpu.sync_copy(idx_hbm.at[wid], idx_v)
        copies = [
            pltpu.async_copy(table_hbm.at[idx_v.at[j]], rows_v.at[j], sem)
            for j in range(K)
        ]
        for c in copies:
            c.wait()
        pltpu.sync_copy(rows_v, out_hbm.at[wid])

    return gather_kernel(table, idx3)


VB = 8192


def _transpose_body(in_ref, out_ref):
    vb = in_ref.shape[1]
    y = in_ref[...].T
    out_ref[...] = jnp.concatenate([y[: vb // 2], y[vb // 2:]], axis=1)


def _transpose_table(tabT):
    V = tabT.shape[1]
    g = pl.cdiv(V, VB)
    return pl.pallas_call(
        _transpose_body,
        grid=(g,),
        in_specs=[pl.BlockSpec((EMB, VB), lambda i: (0, i))],
        out_specs=pl.BlockSpec((VB // 2, 2 * EMB), lambda i: (i, 0)),
        out_shape=jax.ShapeDtypeStruct((g * (VB // 2), 2 * EMB), jnp.float32),
    )(tabT)


def _gru_body(xs_ref, wih_ref, whh_ref, bih_ref, bhh_ref, out_ref, h_ref):
    t = pl.program_id(0)

    @pl.when(t == 0)
    def _():
        h_ref[...] = jnp.zeros_like(h_ref)

    x = xs_ref[0]
    h = h_ref[...]
    gi = jnp.dot(x, wih_ref[...], preferred_element_type=jnp.float32) + bih_ref[...]
    gh = jnp.dot(h, whh_ref[...], preferred_element_type=jnp.float32) + bhh_ref[...]
    r = jax.nn.sigmoid(gi[:, :HID] + gh[:, :HID])
    z = jax.nn.sigmoid(gi[:, HID:2 * HID] + gh[:, HID:2 * HID])
    n = jnp.tanh(gi[:, 2 * HID:] + r * gh[:, 2 * HID:])
    h_new = n + z * (h - n)
    h_ref[...] = h_new

    @pl.when(t == pl.num_programs(0) - 1)
    def _():
        out_ref[0] = h_new


def _gru(xs, wih_t, whh_t, bih, bhh, interpret=False):
    T, B, _ = xs.shape
    return pl.pallas_call(
        _gru_body,
        grid=(T,),
        in_specs=[
            pl.BlockSpec((1, B, EMB), lambda t: (t, 0, 0)),
            pl.BlockSpec((EMB, 3 * HID), lambda t: (0, 0)),
            pl.BlockSpec((HID, 3 * HID), lambda t: (0, 0)),
            pl.BlockSpec((1, 3 * HID), lambda t: (0, 0)),
            pl.BlockSpec((1, 3 * HID), lambda t: (0, 0)),
        ],
        out_specs=pl.BlockSpec((1, B, HID), lambda t: (0, 0, 0)),
        out_shape=jax.ShapeDtypeStruct((1, B, HID), jnp.float32),
        scratch_shapes=[pltpu.VMEM((B, HID), jnp.float32)],
        interpret=interpret,
    )(xs, wih_t, whh_t, bih, bhh)


def kernel(src, emb_table, W_ih, W_hh, b_ih, b_hh):
    B, T = src.shape
    n = B * T
    per_w = n // NW
    k = per_w // CHUNK
    v = src.astype(jnp.int32).T.reshape(-1)
    blk, off = v // VB, v % VB
    half = off // (VB // 2)
    r = blk * VB + 2 * (off % (VB // 2)) + half
    idx3 = r.reshape(NW, k, CHUNK)
    tab16 = _transpose_table(emb_table.T).reshape(-1, EMB)
    rows = _sc_gather(tab16, idx3)
    xs = rows.reshape(T, B, EMB)
    h = _gru(xs, W_ih.T, W_hh.T, b_ih.reshape(1, -1), b_hh.reshape(1, -1))
    return h


# speedup vs baseline: 3.0445x; 1.1565x over previous
import functools

import jax
import jax.numpy as jnp
from jax import lax
from jax.experimental import pallas as pl
from jax.experimental.pallas import tpu as pltpu
from jax.experimental.pallas import tpu_sc as plsc

EMB = 64
HID = 128
NC, NS = 2, 16
NW = NC * NS
CHUNK = 100


def _sc_gather(table, idx3):
    K = idx3.shape[1]
    mesh = plsc.VectorSubcoreMesh(core_axis_name="c", subcore_axis_name="s")

    @functools.partial(
        pl.kernel,
        out_type=jax.ShapeDtypeStruct((NW, K, CHUNK, EMB), jnp.float32),
        mesh=mesh,
        scratch_types=[
            pltpu.VMEM((K, CHUNK), jnp.int32),
            pltpu.VMEM((K, CHUNK, EMB), jnp.float32),
            pltpu.SemaphoreType.DMA,
        ],
        compiler_params=pltpu.CompilerParams(use_tc_tiling_on_sc=False),
    )
    def gather_kernel(table_hbm, idx_hbm, out_hbm, idx_v, rows_v, sem):
        wid = lax.axis_index("s") * NC + lax.axis_index("c")
        pltpu.sync_copy(idx_hbm.at[wid], idx_v)
        copies = [
            pltpu.async_copy(table_hbm.at[idx_v.at[j]], rows_v.at[j], sem)
            for j in range(K)
        ]
        for c in copies:
            c.wait()
        pltpu.sync_copy(rows_v, out_hbm.at[wid])

    return gather_kernel(table, idx3)


VB = 32768


def _transpose_body(in_ref, out_ref):
    vb = in_ref.shape[1]
    y = in_ref[...].T
    out_ref[...] = jnp.concatenate([y[: vb // 2], y[vb // 2:]], axis=1)


def _transpose_table(tabT):
    V = tabT.shape[1]
    g = pl.cdiv(V, VB)
    return pl.pallas_call(
        _transpose_body,
        grid=(g,),
        in_specs=[pl.BlockSpec((EMB, VB), lambda i: (0, i))],
        out_specs=pl.BlockSpec((VB // 2, 2 * EMB), lambda i: (i, 0)),
        out_shape=jax.ShapeDtypeStruct((g * (VB // 2), 2 * EMB), jnp.float32),
    )(tabT)


def _gru_body(xs_ref, wih_ref, whh_ref, bih_ref, bhh_ref, out_ref, h_ref):
    t = pl.program_id(0)

    @pl.when(t == 0)
    def _():
        h_ref[...] = jnp.zeros_like(h_ref)

    x = xs_ref[0]
    h = h_ref[...]
    gi = jnp.dot(x, wih_ref[...], preferred_element_type=jnp.float32) + bih_ref[...]
    gh = jnp.dot(h, whh_ref[...], preferred_element_type=jnp.float32) + bhh_ref[...]
    r = jax.nn.sigmoid(gi[:, :HID] + gh[:, :HID])
    z = jax.nn.sigmoid(gi[:, HID:2 * HID] + gh[:, HID:2 * HID])
    n = jnp.tanh(gi[:, 2 * HID:] + r * gh[:, 2 * HID:])
    h_new = n + z * (h - n)
    h_ref[...] = h_new

    @pl.when(t == pl.num_programs(0) - 1)
    def _():
        out_ref[0] = h_new


def _gru(xs, wih_t, whh_t, bih, bhh, interpret=False):
    T, B, _ = xs.shape
    return pl.pallas_call(
        _gru_body,
        grid=(T,),
        in_specs=[
            pl.BlockSpec((1, B, EMB), lambda t: (t, 0, 0)),
            pl.BlockSpec((EMB, 3 * HID), lambda t: (0, 0)),
            pl.BlockSpec((HID, 3 * HID), lambda t: (0, 0)),
            pl.BlockSpec((1, 3 * HID), lambda t: (0, 0)),
            pl.BlockSpec((1, 3 * HID), lambda t: (0, 0)),
        ],
        out_specs=pl.BlockSpec((1, B, HID), lambda t: (0, 0, 0)),
        out_shape=jax.ShapeDtypeStruct((1, B, HID), jnp.float32),
        scratch_shapes=[pltpu.VMEM((B, HID), jnp.float32)],
        interpret=interpret,
    )(xs, wih_t, whh_t, bih, bhh)


def kernel(src, emb_table, W_ih, W_hh, b_ih, b_hh):
    B, T = src.shape
    n = B * T
    per_w = n // NW
    k = per_w // CHUNK
    v = src.astype(jnp.int32).T.reshape(-1)
    blk, off = v // VB, v % VB
    half = off // (VB // 2)
    r = blk * VB + 2 * (off % (VB // 2)) + half
    idx3 = r.reshape(NW, k, CHUNK)
    tab16 = _transpose_table(emb_table.T).reshape(-1, EMB)
    rows = _sc_gather(tab16, idx3)
    xs = rows.reshape(T, B, EMB)
    h = _gru(xs, W_ih.T, W_hh.T, b_ih.reshape(1, -1), b_hh.reshape(1, -1))
    return h


# speedup vs baseline: 3.2601x; 1.0708x over previous
import functools

import jax
import jax.numpy as jnp
from jax import lax
from jax.experimental import pallas as pl
from jax.experimental.pallas import tpu as pltpu
from jax.experimental.pallas import tpu_sc as plsc

EMB = 64
HID = 128
NC, NS = 2, 16
NW = NC * NS
CHUNK = 100


def _sc_gather(table, idx3):
    K = idx3.shape[1]
    mesh = plsc.VectorSubcoreMesh(core_axis_name="c", subcore_axis_name="s")

    @functools.partial(
        pl.kernel,
        out_type=jax.ShapeDtypeStruct((NW, K, CHUNK, 2 * EMB), jnp.float32),
        mesh=mesh,
        scratch_types=[
            pltpu.VMEM((K, CHUNK), jnp.int32),
            pltpu.VMEM((K, CHUNK, EMB), jnp.float32),
            pltpu.SemaphoreType.DMA,
        ],
        compiler_params=pltpu.CompilerParams(use_tc_tiling_on_sc=False),
    )
    def gather_kernel(table_hbm, idx_hbm, out_hbm, idx_v, rows_v, sem):
        wid = lax.axis_index("s") * NC + lax.axis_index("c")
        pltpu.sync_copy(idx_hbm.at[wid], idx_v)
        copies = [
            pltpu.async_copy(table_hbm.at[idx_v.at[j]], rows_v.at[j], sem)
            for j in range(K)
        ]
        for c in copies:
            c.wait()
        pltpu.sync_copy(rows_v, out_hbm.at[wid, :, :, pl.ds(0, EMB)])

    return gather_kernel(table, idx3)


VB = 32768


def _transpose_body(in_ref, out_ref):
    vb = in_ref.shape[1]
    y = in_ref[...].T
    out_ref[...] = jnp.concatenate([y[: vb // 2], y[vb // 2:]], axis=1)


def _transpose_table(tabT):
    V = tabT.shape[1]
    g = pl.cdiv(V, VB)
    return pl.pallas_call(
        _transpose_body,
        grid=(g,),
        in_specs=[pl.BlockSpec((EMB, VB), lambda i: (0, i))],
        out_specs=pl.BlockSpec((VB // 2, 2 * EMB), lambda i: (i, 0)),
        out_shape=jax.ShapeDtypeStruct((g * (VB // 2), 2 * EMB), jnp.float32),
    )(tabT)


def _gru_body(xs_ref, wih_ref, whh_ref, bih_ref, bhh_ref, out_ref, h_ref):
    t = pl.program_id(0)

    @pl.when(t == 0)
    def _():
        h_ref[...] = jnp.zeros_like(h_ref)

    x = xs_ref[0][:, :EMB]
    h = h_ref[...]
    gi = jnp.dot(x, wih_ref[...], preferred_element_type=jnp.float32) + bih_ref[...]
    gh = jnp.dot(h, whh_ref[...], preferred_element_type=jnp.float32) + bhh_ref[...]
    r = jax.nn.sigmoid(gi[:, :HID] + gh[:, :HID])
    z = jax.nn.sigmoid(gi[:, HID:2 * HID] + gh[:, HID:2 * HID])
    n = jnp.tanh(gi[:, 2 * HID:] + r * gh[:, 2 * HID:])
    h_new = n + z * (h - n)
    h_ref[...] = h_new

    @pl.when(t == pl.num_programs(0) - 1)
    def _():
        out_ref[0] = h_new


def _gru(xs, wih_t, whh_t, bih, bhh, interpret=False):
    T, B, _ = xs.shape
    return pl.pallas_call(
        _gru_body,
        grid=(T,),
        in_specs=[
            pl.BlockSpec((1, B, 2 * EMB), lambda t: (t, 0, 0)),
            pl.BlockSpec((EMB, 3 * HID), lambda t: (0, 0)),
            pl.BlockSpec((HID, 3 * HID), lambda t: (0, 0)),
            pl.BlockSpec((1, 3 * HID), lambda t: (0, 0)),
            pl.BlockSpec((1, 3 * HID), lambda t: (0, 0)),
        ],
        out_specs=pl.BlockSpec((1, B, HID), lambda t: (0, 0, 0)),
        out_shape=jax.ShapeDtypeStruct((1, B, HID), jnp.float32),
        scratch_shapes=[pltpu.VMEM((B, HID), jnp.float32)],
        interpret=interpret,
    )(xs, wih_t, whh_t, bih, bhh)


def kernel(src, emb_table, W_ih, W_hh, b_ih, b_hh):
    B, T = src.shape
    n = B * T
    per_w = n // NW
    k = per_w // CHUNK
    v = src.astype(jnp.int32).T.reshape(-1)
    blk, off = v // VB, v % VB
    half = off // (VB // 2)
    r = blk * VB + 2 * (off % (VB // 2)) + half
    idx3 = r.reshape(NW, k, CHUNK)
    tab16 = _transpose_table(emb_table.T).reshape(-1, EMB)
    rows = _sc_gather(tab16, idx3)
    xs = rows.reshape(T, B, 2 * EMB)
    h = _gru(xs, W_ih.T, W_hh.T, b_ih.reshape(1, -1), b_hh.reshape(1, -1))
    return h


# speedup vs baseline: 3.4805x; 1.0676x over previous
import functools

import jax
import jax.numpy as jnp
from jax import lax
from jax.experimental import pallas as pl
from jax.experimental.pallas import tpu as pltpu
from jax.experimental.pallas import tpu_sc as plsc

EMB = 64
HID = 128
NC, NS = 2, 16
NW = NC * NS
CHUNK = 100


def _sc_gather(table, idx3):
    K = idx3.shape[1]
    mesh = plsc.VectorSubcoreMesh(core_axis_name="c", subcore_axis_name="s")

    @functools.partial(
        pl.kernel,
        out_type=jax.ShapeDtypeStruct((NW, K, CHUNK, 2 * EMB), jnp.float32),
        mesh=mesh,
        scratch_types=[
            pltpu.VMEM((K, CHUNK), jnp.int32),
            pltpu.VMEM((K, CHUNK, EMB), jnp.float32),
            pltpu.SemaphoreType.DMA,
        ],
        compiler_params=pltpu.CompilerParams(use_tc_tiling_on_sc=False),
    )
    def gather_kernel(table_hbm, idx_hbm, out_hbm, idx_v, rows_v, sem):
        wid = lax.axis_index("s") * NC + lax.axis_index("c")
        pltpu.sync_copy(idx_hbm.at[wid], idx_v)
        copies = [
            pltpu.async_copy(table_hbm.at[idx_v.at[j]], rows_v.at[j], sem)
            for j in range(K)
        ]
        for c in copies:
            c.wait()
        pltpu.sync_copy(rows_v, out_hbm.at[wid, :, :, pl.ds(0, EMB)])

    return gather_kernel(table, idx3)


VB = 32768


def _transpose_body(in_ref, out_ref):
    vb = in_ref.shape[1]
    y = in_ref[...].T
    out_ref[...] = jnp.concatenate([y[: vb // 2], y[vb // 2:]], axis=1)


def _transpose_table(tabT):
    V = tabT.shape[1]
    g = pl.cdiv(V, VB)
    return pl.pallas_call(
        _transpose_body,
        grid=(g,),
        in_specs=[pl.BlockSpec((EMB, VB), lambda i: (0, i))],
        out_specs=pl.BlockSpec((VB // 2, 2 * EMB), lambda i: (i, 0)),
        out_shape=jax.ShapeDtypeStruct((g * (VB // 2), 2 * EMB), jnp.float32),
    )(tabT)


UNROLL = 5


def _gru_body(xs_ref, wih_ref, whh_ref, bih_ref, bhh_ref, out_ref, h_ref):
    t = pl.program_id(0)

    @pl.when(t == 0)
    def _():
        h_ref[...] = jnp.zeros_like(h_ref)

    h = h_ref[...]
    for s in range(UNROLL):
        x = xs_ref[s][:, :EMB]
        gi = (jnp.dot(x, wih_ref[...], preferred_element_type=jnp.float32)
              + bih_ref[...])
        gh = (jnp.dot(h, whh_ref[...], preferred_element_type=jnp.float32)
              + bhh_ref[...])
        r = jax.nn.sigmoid(gi[:, :HID] + gh[:, :HID])
        z = jax.nn.sigmoid(gi[:, HID:2 * HID] + gh[:, HID:2 * HID])
        n = jnp.tanh(gi[:, 2 * HID:] + r * gh[:, 2 * HID:])
        h = n + z * (h - n)
    h_ref[...] = h

    @pl.when(t == pl.num_programs(0) - 1)
    def _():
        out_ref[0] = h


def _gru(xs, wih_t, whh_t, bih, bhh, interpret=False):
    T, B, _ = xs.shape
    return pl.pallas_call(
        _gru_body,
        grid=(T // UNROLL,),
        in_specs=[
            pl.BlockSpec((UNROLL, B, 2 * EMB), lambda t: (t, 0, 0)),
            pl.BlockSpec((EMB, 3 * HID), lambda t: (0, 0)),
            pl.BlockSpec((HID, 3 * HID), lambda t: (0, 0)),
            pl.BlockSpec((1, 3 * HID), lambda t: (0, 0)),
            pl.BlockSpec((1, 3 * HID), lambda t: (0, 0)),
        ],
        out_specs=pl.BlockSpec((1, B, HID), lambda t: (0, 0, 0)),
        out_shape=jax.ShapeDtypeStruct((1, B, HID), jnp.float32),
        scratch_shapes=[pltpu.VMEM((B, HID), jnp.float32)],
        interpret=interpret,
    )(xs, wih_t, whh_t, bih, bhh)


def kernel(src, emb_table, W_ih, W_hh, b_ih, b_hh):
    B, T = src.shape
    n = B * T
    per_w = n // NW
    k = per_w // CHUNK
    v = src.astype(jnp.int32).T.reshape(-1)
    blk, off = v // VB, v % VB
    half = off // (VB // 2)
    r = blk * VB + 2 * (off % (VB // 2)) + half
    idx3 = r.reshape(NW, k, CHUNK)
    tab16 = _transpose_table(emb_table.T).reshape(-1, EMB)
    rows = _sc_gather(tab16, idx3)
    xs = rows.reshape(T, B, 2 * EMB)
    h = _gru(xs, W_ih.T, W_hh.T, b_ih.reshape(1, -1), b_hh.reshape(1, -1))
    return h


# speedup vs baseline: 3.5049x; 1.0070x over previous
import functools

import jax
import jax.numpy as jnp
from jax import lax
from jax.experimental import pallas as pl
from jax.experimental.pallas import tpu as pltpu
from jax.experimental.pallas import tpu_sc as plsc

EMB = 64
HID = 128
NC, NS = 2, 16
NW = NC * NS
CHUNK = 100


def _sc_gather(table, idx3):
    K = idx3.shape[1]
    mesh = plsc.VectorSubcoreMesh(core_axis_name="c", subcore_axis_name="s")

    @functools.partial(
        pl.kernel,
        out_type=jax.ShapeDtypeStruct((NW, K, CHUNK, 2 * EMB), jnp.float32),
        mesh=mesh,
        scratch_types=[
            pltpu.VMEM((K, CHUNK), jnp.int32),
            pltpu.VMEM((K, CHUNK, EMB), jnp.float32),
            pltpu.SemaphoreType.DMA,
        ],
        compiler_params=pltpu.CompilerParams(use_tc_tiling_on_sc=False),
    )
    def gather_kernel(table_hbm, idx_hbm, out_hbm, idx_v, rows_v, sem):
        wid = lax.axis_index("s") * NC + lax.axis_index("c")
        pltpu.sync_copy(idx_hbm.at[wid], idx_v)
        copies = [
            pltpu.async_copy(table_hbm.at[idx_v.at[j]], rows_v.at[j], sem)
            for j in range(K)
        ]
        for c in copies:
            c.wait()
        pltpu.sync_copy(rows_v, out_hbm.at[wid, :, :, pl.ds(0, EMB)])

    return gather_kernel(table, idx3)


VB = 32768


def _transpose_body(in_ref, out_ref):
    vb = in_ref.shape[1]
    y = in_ref[...].T
    out_ref[...] = jnp.concatenate([y[: vb // 2], y[vb // 2:]], axis=1)


def _transpose_table(tabT):
    V = tabT.shape[1]
    g = pl.cdiv(V, VB)
    return pl.pallas_call(
        _transpose_body,
        grid=(g,),
        in_specs=[pl.BlockSpec((EMB, VB), lambda i: (0, i))],
        out_specs=pl.BlockSpec((VB // 2, 2 * EMB), lambda i: (i, 0)),
        out_shape=jax.ShapeDtypeStruct((g * (VB // 2), 2 * EMB), jnp.float32),
    )(tabT)


UNROLL = 10


def _gru_body(xs_ref, wih_ref, whh_ref, bih_ref, bhh_ref, out_ref, h_ref):
    t = pl.program_id(0)

    @pl.when(t == 0)
    def _():
        h_ref[...] = jnp.zeros_like(h_ref)

    h = h_ref[...]
    for s in range(UNROLL):
        x = xs_ref[s][:, :EMB]
        gi = (jnp.dot(x, wih_ref[...], preferred_element_type=jnp.float32)
              + bih_ref[...])
        gh = (jnp.dot(h, whh_ref[...], preferred_element_type=jnp.float32)
              + bhh_ref[...])
        r = jax.nn.sigmoid(gi[:, :HID] + gh[:, :HID])
        z = jax.nn.sigmoid(gi[:, HID:2 * HID] + gh[:, HID:2 * HID])
        n = jnp.tanh(gi[:, 2 * HID:] + r * gh[:, 2 * HID:])
        h = n + z * (h - n)
    h_ref[...] = h

    @pl.when(t == pl.num_programs(0) - 1)
    def _():
        out_ref[0] = h


def _gru(xs, wih_t, whh_t, bih, bhh, interpret=False):
    T, B, _ = xs.shape
    return pl.pallas_call(
        _gru_body,
        grid=(T // UNROLL,),
        in_specs=[
            pl.BlockSpec((UNROLL, B, 2 * EMB), lambda t: (t, 0, 0)),
            pl.BlockSpec((EMB, 3 * HID), lambda t: (0, 0)),
            pl.BlockSpec((HID, 3 * HID), lambda t: (0, 0)),
            pl.BlockSpec((1, 3 * HID), lambda t: (0, 0)),
            pl.BlockSpec((1, 3 * HID), lambda t: (0, 0)),
        ],
        out_specs=pl.BlockSpec((1, B, HID), lambda t: (0, 0, 0)),
        out_shape=jax.ShapeDtypeStruct((1, B, HID), jnp.float32),
        scratch_shapes=[pltpu.VMEM((B, HID), jnp.float32)],
        interpret=interpret,
    )(xs, wih_t, whh_t, bih, bhh)


def kernel(src, emb_table, W_ih, W_hh, b_ih, b_hh):
    B, T = src.shape
    n = B * T
    per_w = n // NW
    k = per_w // CHUNK
    v = src.astype(jnp.int32).T.reshape(-1)
    blk, off = v // VB, v % VB
    half = off // (VB // 2)
    r = blk * VB + 2 * (off % (VB // 2)) + half
    idx3 = r.reshape(NW, k, CHUNK)
    tab16 = _transpose_table(emb_table.T).reshape(-1, EMB)
    rows = _sc_gather(tab16, idx3)
    xs = rows.reshape(T, B, 2 * EMB)
    h = _gru(xs, W_ih.T, W_hh.T, b_ih.reshape(1, -1), b_hh.reshape(1, -1))
    return h
